# Initial kernel scaffold; baseline (speedup 1.0000x reference)
import jax
import jax.numpy as jnp
from jax.experimental import pallas as pl


def kernel(x, adj_t, emb, Wl1, bl1, Wr1, Wl2, bl2, Wr2):
    raise NotImplementedError("write your pallas kernel here")


# speedup vs baseline: 4.1619x; 4.1619x over previous
import functools

import jax
import jax.numpy as jnp
from jax import lax
from jax.experimental import pallas as pl
from jax.experimental.pallas import tpu as pltpu
from jax.experimental.pallas import tpu_sc as plsc

N = 10000
D = 128
NC = 2
NS = 16
NW = NC * NS
NPAD = 10240
ROWS_PER_TILE = NPAD // NS
K = 128
BLK = 2000


def _make_sc_pass(d_cols, epw):
    nchunks = epw // K
    mesh = plsc.VectorSubcoreMesh(core_axis_name="c", subcore_axis_name="s")

    @functools.partial(
        pl.kernel,
        out_type=jax.ShapeDtypeStruct((NC * NPAD, d_cols), jnp.float32),
        mesh=mesh,
        scratch_types=[
            pltpu.VMEM((K,), jnp.int32),
            pltpu.VMEM((K,), jnp.int32),
            pltpu.VMEM((K, d_cols), jnp.float32),
            pltpu.VMEM_SHARED((NPAD, d_cols), jnp.float32),
            pltpu.SemaphoreType.DMA,
        ],
        compiler_params=pltpu.CompilerParams(use_tc_tiling_on_sc=False),
    )
    def sc_pass(table_hbm, src_hbm, dst_hbm, out_hbm, sidx_v, didx_v,
                rows_v, acc_sh, sem):
        c = lax.axis_index("c")
        s = lax.axis_index("s")
        wid = s * NC + c

        nseg = d_cols // 16

        def zero_row(i, carry):
            for j in range(nseg):
                rows_v[i, pl.ds(j * 16, 16)] = jnp.zeros((16,), jnp.float32)
            return carry

        lax.fori_loop(0, K, zero_row, 0)
        for r in range(ROWS_PER_TILE // K):
            pltpu.sync_copy(
                rows_v, acc_sh.at[pl.ds(s * ROWS_PER_TILE + r * K, K)])
        plsc.subcore_barrier()

        base = wid * epw

        def body(t, carry):
            off = base + t * K
            pltpu.sync_copy(src_hbm.at[pl.ds(off, K)], sidx_v)
            pltpu.sync_copy(dst_hbm.at[pl.ds(off, K)], didx_v)
            pltpu.async_copy(table_hbm.at[sidx_v], rows_v, sem).wait()
            pltpu.sync_copy(rows_v, acc_sh.at[didx_v], add=True)
            return carry

        lax.fori_loop(0, nchunks, body, 0)
        plsc.subcore_barrier()

        pltpu.sync_copy(
            acc_sh.at[pl.ds(s * ROWS_PER_TILE, ROWS_PER_TILE)],
            out_hbm.at[pl.ds(c * NPAD + s * ROWS_PER_TILE, ROWS_PER_TILE)])

    return sc_pass


def _tc_layer1(acc, emb, wl_t, wr_t, b):
    def body(acc_ref, emb_ref, wl_ref, wr_ref, b_ref, h_ref, rec_ref):
        ssum = acc_ref[0] + acc_ref[1]
        cnt = ssum[:, 128:129]
        rec = 1.0 / jnp.maximum(cnt, 1.0)
        mean = ssum[:, :128] * rec
        h = (jnp.dot(mean, wl_ref[...], preferred_element_type=jnp.float32)
             + jnp.dot(emb_ref[...], wr_ref[...],
                       preferred_element_type=jnp.float32)
             + b_ref[...])
        h_ref[...] = jnp.maximum(h, 0.0)
        rec_ref[...] = rec

    return pl.pallas_call(
        body,
        grid=(N // BLK,),
        in_specs=[
            pl.BlockSpec((2, BLK, 144), lambda i: (0, i, 0)),
            pl.BlockSpec((BLK, D), lambda i: (i, 0)),
            pl.BlockSpec((D, D), lambda i: (0, 0)),
            pl.BlockSpec((D, D), lambda i: (0, 0)),
            pl.BlockSpec((1, D), lambda i: (0, 0)),
        ],
        out_specs=[
            pl.BlockSpec((BLK, D), lambda i: (i, 0)),
            pl.BlockSpec((BLK, 1), lambda i: (i, 0)),
        ],
        out_shape=[
            jax.ShapeDtypeStruct((N, D), jnp.float32),
            jax.ShapeDtypeStruct((N, 1), jnp.float32),
        ],
    )(acc, emb, wl_t, wr_t, b)


def _tc_layer2(acc, rec, h1, wl_t, wr_t, b):
    def body(acc_ref, rec_ref, h1_ref, wl_ref, wr_ref, b_ref, o_ref):
        mean = (acc_ref[0] + acc_ref[1]) * rec_ref[...]
        o_ref[...] = (
            jnp.dot(mean, wl_ref[...], preferred_element_type=jnp.float32)
            + jnp.dot(h1_ref[...], wr_ref[...],
                      preferred_element_type=jnp.float32)
            + b_ref[...])

    return pl.pallas_call(
        body,
        grid=(N // BLK,),
        in_specs=[
            pl.BlockSpec((2, BLK, D), lambda i: (0, i, 0)),
            pl.BlockSpec((BLK, 1), lambda i: (i, 0)),
            pl.BlockSpec((BLK, D), lambda i: (i, 0)),
            pl.BlockSpec((D, D), lambda i: (0, 0)),
            pl.BlockSpec((D, D), lambda i: (0, 0)),
            pl.BlockSpec((1, D), lambda i: (0, 0)),
        ],
        out_specs=pl.BlockSpec((BLK, D), lambda i: (i, 0)),
        out_shape=jax.ShapeDtypeStruct((N, D), jnp.float32),
    )(acc, rec, h1, wl_t, wr_t, b)


def kernel(x, adj_t, emb, Wl1, bl1, Wr1, Wl2, bl2, Wr2):
    src = adj_t[0].astype(jnp.int32)
    dst = adj_t[1].astype(jnp.int32)
    e = src.shape[0]
    epw = -(-e // (NW * K)) * K
    epad = NW * epw
    pad = epad - e
    src_p = jnp.concatenate([src, jnp.zeros((pad,), jnp.int32)])
    dst_p = jnp.concatenate([dst, jnp.full((pad,), N, jnp.int32)])

    table1 = jnp.concatenate(
        [emb, jnp.ones((N, 1), jnp.float32), jnp.zeros((N, 15), jnp.float32)],
        axis=1)

    sc1 = _make_sc_pass(144, epw)
    acc1 = sc1(table1, src_p, dst_p).reshape(NC, NPAD, 144)
    h1, rec = _tc_layer1(acc1, emb, Wl1.T, Wr1.T, bl1.reshape(1, D))

    sc2 = _make_sc_pass(D, epw)
    acc2 = sc2(h1, src_p, dst_p).reshape(NC, NPAD, D)
    out = _tc_layer2(acc2, rec, h1, Wl2.T, Wr2.T, bl2.reshape(1, D))
    return out
